# baseline (speedup 1.0000x reference)
import functools

import jax
import jax.numpy as jnp
from jax import lax
from jax.experimental import pallas as pl
from jax.experimental.pallas import tpu as pltpu
from jax.experimental.pallas import tpu_sc as plsc

N_A = 128
B = 4096
L = 200
NW = 32
RPW = B // NW
NG = (L + 15) // 16


def _sc_hist_body(x_hbm, out_hbm, x_v, cnt_v, sem_a, sem_b, sem_o):
    wid = lax.axis_index("s") * 2 + lax.axis_index("c")
    RH = RPW // 2
    HL = RH * L
    HC = RH * N_A
    xbase0 = wid * (RPW * L)
    obase0 = wid * (RPW * N_A)

    in_a = pltpu.async_copy(x_hbm.at[pl.ds(xbase0, HL)], x_v.at[pl.ds(0, HL)], sem_a)
    in_b = pltpu.async_copy(
        x_hbm.at[pl.ds(xbase0 + HL, HL)], x_v.at[pl.ds(HL, HL)], sem_b
    )

    zeros16 = jnp.zeros((16,), jnp.float32)

    @plsc.parallel_loop(0, RPW * N_A // 16, unroll=16)
    def _zero(i):
        cnt_v[pl.ds(i * 16, 16)] = zeros16

    lane = lax.iota(jnp.int32, 16)
    m_first = lane >= 1
    m_last = lane < (L - (NG - 1) * 16)
    ones16 = jnp.ones((16,), jnp.float32)

    def scatter_row(r):
        xbase = r * L
        row_ref = cnt_v.at[pl.ds(r * N_A, N_A)]
        for g in range(NG):
            vals = x_v[pl.ds(xbase + g * 16, 16)]
            if g == 0:
                mask = m_first
            elif g == NG - 1:
                mask = m_last
            else:
                mask = None
            plsc.addupdate_scatter(row_ref, [vals], ones16, mask=mask)

    in_a.wait()

    @plsc.parallel_loop(0, RH, unroll=8)
    def _row_a(r):
        scatter_row(r)

    out_a = pltpu.async_copy(
        cnt_v.at[pl.ds(0, HC)], out_hbm.at[pl.ds(obase0, HC)], sem_o
    )
    in_b.wait()

    @plsc.parallel_loop(RH, RPW, unroll=8)
    def _row_b(r):
        scatter_row(r)

    out_a.wait()
    pltpu.sync_copy(cnt_v.at[pl.ds(HC, HC)], out_hbm.at[pl.ds(obase0 + HC, HC)])


@functools.partial(
    pl.kernel,
    mesh=plsc.VectorSubcoreMesh(core_axis_name="c", subcore_axis_name="s"),
    compiler_params=pltpu.CompilerParams(needs_layout_passes=False),
    out_type=jax.ShapeDtypeStruct((B * N_A,), jnp.float32),
    scratch_types=[
        pltpu.VMEM((RPW * L,), jnp.int32),
        pltpu.VMEM((RPW * N_A,), jnp.float32),
        pltpu.SemaphoreType.DMA,
        pltpu.SemaphoreType.DMA,
        pltpu.SemaphoreType.DMA,
    ],
)
def _sc_hist(x_hbm, out_hbm, x_v, cnt_v, sem_a, sem_b, sem_o):
    _sc_hist_body(x_hbm, out_hbm, x_v, cnt_v, sem_a, sem_b, sem_o)


def _tc_body(c_ref, w_ref, b_ref, o_ref):
    o_ref[...] = (
        jax.lax.dot_general(
            c_ref[...], w_ref[...], (((1,), (1,)), ((), ())),
            preferred_element_type=jnp.float32,
        )
        + b_ref[...]
    )


def _tc_linear(counts, w, b2):
    brow = 2048
    return pl.pallas_call(
        _tc_body,
        grid=(B // brow,),
        in_specs=[
            pl.BlockSpec((brow, N_A), lambda i: (i, 0)),
            pl.BlockSpec((N_A, N_A), lambda i: (0, 0)),
            pl.BlockSpec((1, N_A), lambda i: (0, 0)),
        ],
        out_specs=pl.BlockSpec((brow, N_A), lambda i: (i, 0)),
        out_shape=jax.ShapeDtypeStruct((B, N_A), jnp.float32),
    )(counts, w, b2)


def kernel(x, W, b):
    x1 = x.astype(jnp.int32).reshape(B * L)
    counts = _sc_hist(x1).reshape(B, N_A)
    return _tc_linear(counts, W, b.reshape(1, N_A))


# speedup vs baseline: 1.0037x; 1.0037x over previous
import functools

import jax
import jax.numpy as jnp
from jax import lax
from jax.experimental import pallas as pl
from jax.experimental.pallas import tpu as pltpu
from jax.experimental.pallas import tpu_sc as plsc

N_A = 128
B = 4096
L = 200
NW = 32
RPW = B // NW
NG = (L + 15) // 16


def _sc_hist_body(x_hbm, out_hbm, x_v, cnt_v, sem_a, sem_b, sem_o):
    wid = lax.axis_index("s") * 2 + lax.axis_index("c")
    RH = RPW // 2
    HL = RH * L
    HC = RH * N_A
    xbase0 = wid * (RPW * L)
    obase0 = wid * (RPW * N_A)

    in_a = pltpu.async_copy(x_hbm.at[pl.ds(xbase0, HL)], x_v.at[pl.ds(0, HL)], sem_a)
    in_b = pltpu.async_copy(
        x_hbm.at[pl.ds(xbase0 + HL, HL)], x_v.at[pl.ds(HL, HL)], sem_b
    )

    zeros16 = jnp.zeros((16,), jnp.float32)

    @plsc.parallel_loop(0, RPW * N_A // 16, unroll=16)
    def _zero(i):
        cnt_v[pl.ds(i * 16, 16)] = zeros16

    lane = lax.iota(jnp.int32, 16)
    m_first = lane >= 1
    m_last = lane < (L - (NG - 1) * 16)
    ones16 = jnp.ones((16,), jnp.float32)

    def scatter_row(r):
        xbase = r * L
        row_ref = cnt_v.at[pl.ds(r * N_A, N_A)]
        for g in range(NG):
            vals = x_v[pl.ds(xbase + g * 16, 16)]
            if g == 0:
                mask = m_first
            elif g == NG - 1:
                mask = m_last
            else:
                mask = None
            plsc.addupdate_scatter(row_ref, [vals], ones16, mask=mask)

    in_a.wait()

    @plsc.parallel_loop(0, RH, unroll=8)
    def _row_a(r):
        scatter_row(r)

    out_a = pltpu.async_copy(
        cnt_v.at[pl.ds(0, HC)], out_hbm.at[pl.ds(obase0, HC)], sem_o
    )
    in_b.wait()

    @plsc.parallel_loop(RH, RPW, unroll=8)
    def _row_b(r):
        scatter_row(r)

    out_a.wait()
    pltpu.sync_copy(cnt_v.at[pl.ds(HC, HC)], out_hbm.at[pl.ds(obase0 + HC, HC)])


@functools.lru_cache(maxsize=1)
def _sc_hist():
    return pl.kernel(
        _sc_hist_body,
        mesh=plsc.VectorSubcoreMesh(core_axis_name="c", subcore_axis_name="s"),
        compiler_params=pltpu.CompilerParams(needs_layout_passes=False),
        out_type=jax.ShapeDtypeStruct((B * N_A,), jnp.float32),
        scratch_types=[
            pltpu.VMEM((RPW * L,), jnp.int32),
            pltpu.VMEM((RPW * N_A,), jnp.float32),
            pltpu.SemaphoreType.DMA,
            pltpu.SemaphoreType.DMA,
            pltpu.SemaphoreType.DMA,
        ],
    )


def _tc_body(c_ref, w_ref, b_ref, o_ref):
    o_ref[...] = (
        jax.lax.dot_general(
            c_ref[...], w_ref[...], (((1,), (1,)), ((), ())),
            preferred_element_type=jnp.float32,
        )
        + b_ref[...]
    )


def _tc_linear(counts, w, b2):
    brow = 2048
    return pl.pallas_call(
        _tc_body,
        grid=(B // brow,),
        in_specs=[
            pl.BlockSpec((brow, N_A), lambda i: (i, 0)),
            pl.BlockSpec((N_A, N_A), lambda i: (0, 0)),
            pl.BlockSpec((1, N_A), lambda i: (0, 0)),
        ],
        out_specs=pl.BlockSpec((brow, N_A), lambda i: (i, 0)),
        out_shape=jax.ShapeDtypeStruct((B, N_A), jnp.float32),
    )(counts, w, b2)


def kernel(x, W, b):
    x1 = x.astype(jnp.int32).reshape(B * L)
    counts = _sc_hist()(x1).reshape(B, N_A)
    return _tc_linear(counts, W, b.reshape(1, N_A))
